# baseline (speedup 1.0000x reference)
import functools
import jax
import jax.numpy as jnp
from jax import lax
from jax.experimental import pallas as pl
from jax.experimental.pallas import tpu as pltpu
from jax.experimental.pallas import tpu_sc as plsc

N = 10000
E = 320000
R = 9
DIN = 5
D = 128
G = 64

NC = 2
NS = 16
NW = NC * NS
CH = 128
KCH = 80
EPT = CH * KCH
EPAD = EPT * NW
NR = N * R
ROWS_A = 5632
NRP = ROWS_A * NS
ROWS_C = 632
NP = ROWS_C * NS

_mesh = plsc.VectorSubcoreMesh(core_axis_name="c", subcore_axis_name="s",
                               num_cores=NC, num_subcores=NS)
_sc_params = pltpu.CompilerParams(use_tc_tiling_on_sc=False,
                                  needs_layout_passes=False)


@functools.partial(
    pl.kernel,
    out_type=jax.ShapeDtypeStruct((NC, NRP, 16), jnp.float32),
    mesh=_mesh,
    compiler_params=_sc_params,
    scratch_types=[
        pltpu.VMEM((EPT,), jnp.int32),
        pltpu.VMEM((KCH, CH), jnp.int32),
        pltpu.VMEM((CH, 16), jnp.float32),
        pltpu.VMEM((CH, 16), jnp.float32),
        pltpu.VMEM_SHARED((NRP, 16), jnp.float32),
        pltpu.SemaphoreType.DMA,
        pltpu.SemaphoreType.DMA,
    ],
)
def _edge_sums1(xpad_hbm, src_hbm, seg_hbm, zeros_hbm, out_hbm,
                srcbuf, segbuf, rows0, rows1, acc, semA, semB):
    c = lax.axis_index("c")
    s = lax.axis_index("s")
    wid = c * NS + s
    base = wid * EPT

    pltpu.sync_copy(src_hbm.at[pl.ds(base, EPT)], srcbuf)
    pltpu.sync_copy(seg_hbm.at[pl.ds(wid * KCH, KCH)], segbuf)

    pltpu.sync_copy(zeros_hbm.at[pl.ds(s * ROWS_A, ROWS_A)],
                    acc.at[pl.ds(s * ROWS_A, ROWS_A)])
    plsc.subcore_barrier()

    pltpu.async_copy(xpad_hbm.at[srcbuf.at[pl.ds(0, CH)]], rows0, semA)

    @pl.loop(0, KCH, step=2)
    def _pair(j):
        pltpu.async_copy(xpad_hbm.at[srcbuf.at[pl.ds((j + 1) * CH, CH)]],
                         rows1, semB)
        pltpu.make_async_copy(xpad_hbm.at[pl.ds(0, CH)], rows0, semA).wait()
        pltpu.sync_copy(rows0, acc.at[segbuf.at[j]], add=True)

        @pl.when(j + 2 < KCH)
        def _():
            pltpu.async_copy(xpad_hbm.at[srcbuf.at[pl.ds((j + 2) * CH, CH)]],
                             rows0, semA)

        pltpu.make_async_copy(xpad_hbm.at[pl.ds(0, CH)], rows1, semB).wait()
        pltpu.sync_copy(rows1, acc.at[segbuf.at[j + 1]], add=True)

    plsc.subcore_barrier()
    pltpu.sync_copy(acc.at[pl.ds(s * ROWS_A, ROWS_A)],
                    out_hbm.at[c, pl.ds(s * ROWS_A, ROWS_A)])


@functools.partial(
    pl.kernel,
    out_type=jax.ShapeDtypeStruct((EPAD,), jnp.float32),
    mesh=_mesh,
    compiler_params=_sc_params,
    scratch_types=[
        pltpu.VMEM((NR + 16,), jnp.float32),
        pltpu.VMEM((EPT,), jnp.int32),
        pltpu.VMEM((EPT,), jnp.float32),
    ],
)
def _edge_alpha(ic_hbm, seg_hbm, out_hbm, ictab, segbuf, abuf):
    c = lax.axis_index("c")
    s = lax.axis_index("s")
    wid = c * NS + s
    base = wid * EPT
    pltpu.sync_copy(ic_hbm, ictab)
    pltpu.sync_copy(seg_hbm.at[pl.ds(base, EPT)], segbuf)

    @pl.loop(0, EPT // 16)
    def _grp(j):
        segv = segbuf[pl.ds(j * 16, 16)]
        abuf[pl.ds(j * 16, 16)] = plsc.load_gather(ictab, [segv])

    pltpu.sync_copy(abuf, out_hbm.at[pl.ds(base, EPT)])


CH2 = 64
KC2 = EPT // CH2
NBUF = 4


@functools.partial(
    pl.kernel,
    out_type=jax.ShapeDtypeStruct((NC, NP, D), jnp.float32),
    mesh=_mesh,
    compiler_params=_sc_params,
    scratch_types=[
        pltpu.VMEM((EPT,), jnp.float32),
        pltpu.VMEM((KC2, CH2), jnp.int32),
        pltpu.VMEM((2 * NBUF, CH2), jnp.int32),
        pltpu.VMEM((CH2, D // 2), jnp.int32),
        pltpu.VMEM((CH2, D // 2), jnp.int32),
        pltpu.VMEM((CH2, D // 2), jnp.int32),
        pltpu.VMEM((CH2, D // 2), jnp.int32),
        pltpu.VMEM((CH2, D), jnp.float32),
        pltpu.VMEM_SHARED((NP, D), jnp.float32),
        [pltpu.SemaphoreType.DMA] * NBUF,
        [pltpu.SemaphoreType.DMA] * (2 * NBUF),
    ],
)
def _edge_sums2(t2_hbm, tix_hbm, dst_hbm, alpha_hbm, zeros_hbm, out_hbm,
                abuf, dstbuf, tixr, rows0, rows1, rows2, rows3, work, acc,
                gsems, msems):
    c = lax.axis_index("c")
    s = lax.axis_index("s")
    wid = c * NS + s
    cbase = wid * KC2
    rows = [rows0, rows1, rows2, rows3]

    pltpu.sync_copy(alpha_hbm.at[pl.ds(wid * EPT, EPT)], abuf)
    pltpu.sync_copy(dst_hbm.at[wid], dstbuf)
    pltpu.sync_copy(zeros_hbm.at[pl.ds(s * ROWS_C, ROWS_C)],
                    acc.at[pl.ds(s * ROWS_C, ROWS_C)])
    plsc.subcore_barrier()

    def _scale_scatter(rb, i):
        himask = jnp.full((16,), -65536, jnp.int32)

        @pl.loop(0, CH2 // 16)
        def _grp(g):
            av = abuf[pl.ds(i * CH2 + g * 16, 16)]
            for l in range(16):
                e = g * 16 + l
                a = av[l]
                for k in range(D // 32):
                    w = rb[e, pl.ds(k * 16, 16)]
                    lo = plsc.bitcast(lax.shift_left(w, 16), jnp.float32)
                    hi = plsc.bitcast(jnp.bitwise_and(w, himask), jnp.float32)
                    work[e, pl.ds(k * 32, 16)] = lo * a
                    work[e, pl.ds(k * 32 + 16, 16)] = hi * a

        pltpu.sync_copy(work, acc.at[dstbuf.at[i]], add=True)

    for b in range(NBUF):
        pltpu.sync_copy(tix_hbm.at[cbase + b], tixr.at[b])
    for b in range(NBUF):
        pltpu.async_copy(tix_hbm.at[cbase + NBUF + b], tixr.at[NBUF + b],
                         msems[NBUF + b])
    for b in range(NBUF):
        pltpu.async_copy(t2_hbm.at[tixr.at[b]], rows[b], gsems[b])

    @pl.loop(0, KC2, step=2 * NBUF)
    def _oct(j):
        for half in range(2):
            lo = half * NBUF
            hi = NBUF - lo
            for b in range(NBUF):
                i = j + lo + b
                slot = lo + b
                pltpu.make_async_copy(t2_hbm.at[pl.ds(0, CH2)], rows[b],
                                      gsems[b]).wait()
                _scale_scatter(rows[b], i)

                @pl.when(i + 2 * NBUF < KC2)
                def _():
                    pltpu.async_copy(tix_hbm.at[cbase + i + 2 * NBUF],
                                     tixr.at[slot], msems[slot])

                @pl.when(i + NBUF < KC2)
                def _():
                    nslot = hi + b
                    pltpu.make_async_copy(tix_hbm.at[cbase], tixr.at[nslot],
                                          msems[nslot]).wait()
                    pltpu.async_copy(t2_hbm.at[tixr.at[nslot]], rows[b],
                                     gsems[b])

    plsc.subcore_barrier()
    pltpu.sync_copy(acc.at[pl.ds(s * ROWS_C, ROWS_C)],
                    out_hbm.at[c, pl.ds(s * ROWS_C, ROWS_C)])


NB = 1000


def _dense1_body(sp_ref, x_ref, p_ref, w1_ref, r1_ref, b1_ref, w2_ref,
                 qp_ref, h1_ref, t2_ref, ict_ref):
    S = sp_ref[0] + sp_ref[1]
    cnt = S @ p_ref[...]
    ic = 1.0 / jnp.maximum(cnt, 1.0)
    mean = S * ic
    h1 = jnp.maximum(mean @ w1_ref[...] + x_ref[...] @ r1_ref[...]
                     + b1_ref[...], 0.0)
    h1_ref[...] = h1
    t2_ref[...] = h1 @ w2_ref[...]
    ict_ref[...] = ic @ qp_ref[...]


_dense1 = pl.pallas_call(
    _dense1_body,
    grid=(N // NB,),
    in_specs=[
        pl.BlockSpec((2, NB, R * 16), lambda i: (0, i, 0)),
        pl.BlockSpec((NB, 16), lambda i: (i, 0)),
        pl.BlockSpec((R * 16, R * 16), lambda i: (0, 0)),
        pl.BlockSpec((R * 16, D), lambda i: (0, 0)),
        pl.BlockSpec((16, D), lambda i: (0, 0)),
        pl.BlockSpec((1, D), lambda i: (0, 0)),
        pl.BlockSpec((D, R * D), lambda i: (0, 0)),
        pl.BlockSpec((R * 16, 16), lambda i: (0, 0)),
    ],
    out_specs=[
        pl.BlockSpec((NB, D), lambda i: (i, 0)),
        pl.BlockSpec((NB, R * D), lambda i: (i, 0)),
        pl.BlockSpec((NB, 16), lambda i: (i, 0)),
    ],
    out_shape=[
        jax.ShapeDtypeStruct((N, D), jnp.float32),
        jax.ShapeDtypeStruct((N, R * D), jnp.float32),
        jax.ShapeDtypeStruct((N, 16), jnp.float32),
    ],
)


def _dense2_body(ap_ref, h1_ref, r2_ref, b2_ref, h2_ref):
    h2_ref[...] = jnp.maximum(ap_ref[0] + ap_ref[1]
                              + h1_ref[...] @ r2_ref[...] + b2_ref[...], 0.0)


_dense2 = pl.pallas_call(
    _dense2_body,
    grid=(N // NB,),
    in_specs=[
        pl.BlockSpec((2, NB, D), lambda i: (0, i, 0)),
        pl.BlockSpec((NB, D), lambda i: (i, 0)),
        pl.BlockSpec((D, D), lambda i: (0, 0)),
        pl.BlockSpec((1, D), lambda i: (0, 0)),
    ],
    out_specs=pl.BlockSpec((NB, D), lambda i: (i, 0)),
    out_shape=jax.ShapeDtypeStruct((N, D), jnp.float32),
)


def _pool_body(h2_ref, b_ref, gw_ref, fcw_ref, fcb_ref, out_ref):
    h2 = h2_ref[...]
    gate = h2 @ gw_ref[...]
    gate = gate[:, 0:1]
    onehot = (b_ref[...] == lax.broadcasted_iota(jnp.int32, (1, G), 1))
    oh = onehot.astype(jnp.float32)
    gmax = jnp.max(jnp.where(onehot, gate, -jnp.inf), axis=0, keepdims=True)
    gmaxn = jnp.sum(oh * gmax, axis=1, keepdims=True)
    ge = jnp.exp(gate - gmaxn)
    denom = jnp.sum(oh * ge, axis=0, keepdims=True)
    denomn = jnp.sum(oh * denom, axis=1, keepdims=True)
    w = ge / denomn
    pooled = lax.dot_general(oh, h2 * w, (((0,), (0,)), ((), ())))
    out_ref[...] = pooled @ fcw_ref[...] + fcb_ref[...]


_pool = pl.pallas_call(
    _pool_body,
    out_shape=jax.ShapeDtypeStruct((G, D), jnp.float32),
)


def kernel(x, edge_index, edge_type, batch, W1, root1, b1, W2, root2, b2,
           gate_W, gate_b, fc_W, fc_b):
    src = edge_index[0]
    dst = edge_index[1]
    seg = dst * R + edge_type
    npad = EPAD - E

    xpad = jnp.zeros((N + 1, 16), jnp.float32)
    xpad = xpad.at[:N, :DIN].set(x).at[:N, DIN].set(1.0)

    src_p = jnp.concatenate([src, jnp.full((npad,), N, jnp.int32)])
    seg_p = jnp.concatenate([seg, jnp.zeros((npad,), jnp.int32)])
    zeros_a = jnp.zeros((NRP, 16), jnp.float32)

    s2 = _edge_sums1(xpad, src_p, seg_p.reshape(NW * KCH, CH), zeros_a)
    sparts = s2[:, :NR].reshape(2, N, R * 16)

    i144 = jnp.arange(R * 16)
    P = (i144[:, None] == ((i144 // 16) * 16 + DIN)[None, :]).astype(jnp.float32)
    Qp = (i144[:, None] == (jnp.arange(16) * 16 + DIN)[None, :]).astype(jnp.float32)
    W1p = jnp.zeros((R, 16, D), jnp.float32).at[:, :DIN, :].set(W1)
    root1p = jnp.zeros((16, D), jnp.float32).at[:DIN, :].set(root1)
    W2f = jnp.transpose(W2, (1, 0, 2)).reshape(D, R * D)

    h1, t2n, ict = _dense1(sparts, xpad[:N], P, W1p.reshape(R * 16, D),
                           root1p, b1[None], W2f, Qp)
    t2 = lax.bitcast_convert_type(
        t2n.astype(jnp.bfloat16).reshape(N, (R * D) // 2, 2),
        jnp.int32).reshape(NR, D // 2)

    ic_pad = jnp.concatenate([ict[:, :R].reshape(NR),
                              jnp.zeros((16,), jnp.float32)])
    aseg_p = jnp.concatenate([seg, jnp.full((npad,), NR, jnp.int32)])
    alpha = _edge_alpha(ic_pad, aseg_p)

    tix_p = jnp.concatenate([src * R + edge_type, jnp.zeros((npad,), jnp.int32)])
    dst_p = jnp.concatenate([dst, jnp.zeros((npad,), jnp.int32)])
    zeros_c = jnp.zeros((NP, D), jnp.float32)

    a2 = _edge_sums2(t2, tix_p.reshape(NW * KC2, CH2),
                     dst_p.reshape(NW, KC2, CH2), alpha, zeros_c)

    kd = jnp.arange(D)
    m = kd % 32
    perm = (kd // 32) * 32 + jnp.where(m < 16, 2 * m, 2 * (m - 16) + 1)
    h2 = _dense2(a2[:, :N], h1, root2[:, perm], b2[perm][None])

    gWp = jnp.zeros((D, 8), jnp.float32).at[:, 0].set(gate_W[perm, 0])
    return _pool(h2, batch.reshape(N, 1), gWp, fc_W[perm, :], fc_b[None])


# speedup vs baseline: 1.7447x; 1.7447x over previous
import functools
import jax
import jax.numpy as jnp
from jax import lax
from jax.experimental import pallas as pl
from jax.experimental.pallas import tpu as pltpu
from jax.experimental.pallas import tpu_sc as plsc

N = 10000
E = 320000
R = 9
DIN = 5
D = 128
G = 64

NC = 2
NS = 16
NW = NC * NS
CH = 128
KCH = 80
EPT = CH * KCH
EPAD = EPT * NW
NR = N * R
ROWS_A = 5632
NRP = ROWS_A * NS
ROWS_C = 632
NP = ROWS_C * NS

_mesh = plsc.VectorSubcoreMesh(core_axis_name="c", subcore_axis_name="s",
                               num_cores=NC, num_subcores=NS)
_sc_params = pltpu.CompilerParams(use_tc_tiling_on_sc=False,
                                  needs_layout_passes=False)


@functools.partial(
    pl.kernel,
    out_type=jax.ShapeDtypeStruct((NC, NRP, 16), jnp.float32),
    mesh=_mesh,
    compiler_params=_sc_params,
    scratch_types=[
        pltpu.VMEM((EPT,), jnp.int32),
        pltpu.VMEM((KCH, CH), jnp.int32),
        pltpu.VMEM((CH, 16), jnp.float32),
        pltpu.VMEM((CH, 16), jnp.float32),
        pltpu.VMEM_SHARED((NRP, 16), jnp.float32),
        pltpu.SemaphoreType.DMA,
        pltpu.SemaphoreType.DMA,
    ],
)
def _edge_sums1(xpad_hbm, src_hbm, seg_hbm, zeros_hbm, out_hbm,
                srcbuf, segbuf, rows0, rows1, acc, semA, semB):
    c = lax.axis_index("c")
    s = lax.axis_index("s")
    wid = c * NS + s
    base = wid * EPT

    pltpu.sync_copy(src_hbm.at[pl.ds(base, EPT)], srcbuf)
    pltpu.sync_copy(seg_hbm.at[pl.ds(wid * KCH, KCH)], segbuf)

    pltpu.sync_copy(zeros_hbm.at[pl.ds(s * ROWS_A, ROWS_A)],
                    acc.at[pl.ds(s * ROWS_A, ROWS_A)])
    plsc.subcore_barrier()

    pltpu.async_copy(xpad_hbm.at[srcbuf.at[pl.ds(0, CH)]], rows0, semA)

    @pl.loop(0, KCH, step=2)
    def _pair(j):
        pltpu.async_copy(xpad_hbm.at[srcbuf.at[pl.ds((j + 1) * CH, CH)]],
                         rows1, semB)
        pltpu.make_async_copy(xpad_hbm.at[pl.ds(0, CH)], rows0, semA).wait()
        pltpu.sync_copy(rows0, acc.at[segbuf.at[j]], add=True)

        @pl.when(j + 2 < KCH)
        def _():
            pltpu.async_copy(xpad_hbm.at[srcbuf.at[pl.ds((j + 2) * CH, CH)]],
                             rows0, semA)

        pltpu.make_async_copy(xpad_hbm.at[pl.ds(0, CH)], rows1, semB).wait()
        pltpu.sync_copy(rows1, acc.at[segbuf.at[j + 1]], add=True)

    plsc.subcore_barrier()
    pltpu.sync_copy(acc.at[pl.ds(s * ROWS_A, ROWS_A)],
                    out_hbm.at[c, pl.ds(s * ROWS_A, ROWS_A)])


@functools.partial(
    pl.kernel,
    out_type=jax.ShapeDtypeStruct((EPAD,), jnp.float32),
    mesh=_mesh,
    compiler_params=_sc_params,
    scratch_types=[
        pltpu.VMEM((NR + 16,), jnp.float32),
        pltpu.VMEM((EPT,), jnp.int32),
        pltpu.VMEM((EPT,), jnp.float32),
    ],
)
def _edge_alpha(ic_hbm, seg_hbm, out_hbm, ictab, segbuf, abuf):
    c = lax.axis_index("c")
    s = lax.axis_index("s")
    wid = c * NS + s
    base = wid * EPT
    pltpu.sync_copy(ic_hbm, ictab)
    pltpu.sync_copy(seg_hbm.at[pl.ds(base, EPT)], segbuf)

    @pl.loop(0, EPT // 16)
    def _grp(j):
        segv = segbuf[pl.ds(j * 16, 16)]
        abuf[pl.ds(j * 16, 16)] = plsc.load_gather(ictab, [segv])

    pltpu.sync_copy(abuf, out_hbm.at[pl.ds(base, EPT)])


CH2 = 64
KC2 = EPT // CH2
NBUF = 4


@functools.partial(
    pl.kernel,
    out_type=jax.ShapeDtypeStruct((NC, NP, D), jnp.float32),
    mesh=_mesh,
    compiler_params=_sc_params,
    scratch_types=[
        pltpu.VMEM((EPT,), jnp.float32),
        pltpu.VMEM((KC2, CH2), jnp.int32),
        pltpu.VMEM((2 * NBUF, CH2), jnp.int32),
        pltpu.VMEM((CH2, D // 2), jnp.int32),
        pltpu.VMEM((CH2, D // 2), jnp.int32),
        pltpu.VMEM((CH2, D // 2), jnp.int32),
        pltpu.VMEM((CH2, D // 2), jnp.int32),
        pltpu.VMEM((CH2, D), jnp.float32),
        pltpu.VMEM_SHARED((NP, D), jnp.float32),
        [pltpu.SemaphoreType.DMA] * NBUF,
        [pltpu.SemaphoreType.DMA] * (2 * NBUF),
    ],
)
def _edge_sums2(t2_hbm, tix_hbm, dst_hbm, alpha_hbm, zeros_hbm, out_hbm,
                abuf, dstbuf, tixr, rows0, rows1, rows2, rows3, work, acc,
                gsems, msems):
    c = lax.axis_index("c")
    s = lax.axis_index("s")
    wid = c * NS + s
    cbase = wid * KC2
    rows = [rows0, rows1, rows2, rows3]

    pltpu.sync_copy(alpha_hbm.at[pl.ds(wid * EPT, EPT)], abuf)
    pltpu.sync_copy(dst_hbm.at[wid], dstbuf)
    pltpu.sync_copy(zeros_hbm.at[pl.ds(s * ROWS_C, ROWS_C)],
                    acc.at[pl.ds(s * ROWS_C, ROWS_C)])
    plsc.subcore_barrier()

    def _scale_scatter(rb, i):
        himask = jnp.full((16,), -65536, jnp.int32)

        @pl.loop(0, CH2 // 16)
        def _grp(g):
            av = abuf[pl.ds(i * CH2 + g * 16, 16)]
            for l in range(16):
                e = g * 16 + l
                a = av[l]
                for k in range(D // 32):
                    w = rb[e, pl.ds(k * 16, 16)]
                    lo = plsc.bitcast(lax.shift_left(w, 16), jnp.float32)
                    hi = plsc.bitcast(jnp.bitwise_and(w, himask), jnp.float32)
                    work[e, pl.ds(k * 32, 16)] = lo * a
                    work[e, pl.ds(k * 32 + 16, 16)] = hi * a

        pltpu.sync_copy(work, acc.at[dstbuf.at[i]], add=True)

    for b in range(NBUF):
        pltpu.sync_copy(tix_hbm.at[cbase + b], tixr.at[b])
    for b in range(NBUF):
        pltpu.async_copy(tix_hbm.at[cbase + NBUF + b], tixr.at[NBUF + b],
                         msems[NBUF + b])
    for b in range(NBUF):
        pltpu.async_copy(t2_hbm.at[tixr.at[b]], rows[b], gsems[b])

    @pl.loop(0, KC2, step=2 * NBUF)
    def _oct(j):
        for half in range(2):
            lo = half * NBUF
            hi = NBUF - lo
            for b in range(NBUF):
                i = j + lo + b
                slot = lo + b
                pltpu.make_async_copy(t2_hbm.at[pl.ds(0, CH2)], rows[b],
                                      gsems[b]).wait()
                _scale_scatter(rows[b], i)

                @pl.when(i + 2 * NBUF < KC2)
                def _():
                    pltpu.async_copy(tix_hbm.at[cbase + i + 2 * NBUF],
                                     tixr.at[slot], msems[slot])

                @pl.when(i + NBUF < KC2)
                def _():
                    nslot = hi + b
                    pltpu.make_async_copy(tix_hbm.at[cbase], tixr.at[nslot],
                                          msems[nslot]).wait()
                    pltpu.async_copy(t2_hbm.at[tixr.at[nslot]], rows[b],
                                     gsems[b])

    plsc.subcore_barrier()
    pltpu.sync_copy(acc.at[pl.ds(s * ROWS_C, ROWS_C)],
                    out_hbm.at[c, pl.ds(s * ROWS_C, ROWS_C)])


NB = 1000


def _round_bf16_bits(x):
    b = lax.bitcast_convert_type(x, jnp.int32)
    rnd = jnp.bitwise_and(lax.shift_right_logical(b, 16), 1) + 0x7FFF
    return b + rnd


def _dense1_body(sp_ref, x_ref, p_ref, w1_ref, r1_ref, b1_ref, w2_ref,
                 qp_ref, h1_ref, t2_ref, ict_ref):
    S = sp_ref[0] + sp_ref[1]
    cnt = S @ p_ref[...]
    ic = 1.0 / jnp.maximum(cnt, 1.0)
    mean = S * ic
    h1 = jnp.maximum(mean @ w1_ref[...] + x_ref[...] @ r1_ref[...]
                     + b1_ref[...], 0.0)
    h1_ref[...] = h1
    t2f = h1 @ w2_ref[...]
    parts = []
    for r in range(R):
        lo = lax.shift_right_logical(
            _round_bf16_bits(t2f[:, r * D:r * D + D // 2]), 16)
        hi = jnp.bitwise_and(
            _round_bf16_bits(t2f[:, r * D + D // 2:(r + 1) * D]), -65536)
        parts.append(jnp.bitwise_or(hi, lo))
    t2_ref[...] = jnp.concatenate(parts, axis=1)
    ict_ref[...] = ic @ qp_ref[...]


_dense1 = pl.pallas_call(
    _dense1_body,
    grid=(N // NB,),
    in_specs=[
        pl.BlockSpec((2, NB, R * 16), lambda i: (0, i, 0)),
        pl.BlockSpec((NB, 16), lambda i: (i, 0)),
        pl.BlockSpec((R * 16, R * 16), lambda i: (0, 0)),
        pl.BlockSpec((R * 16, D), lambda i: (0, 0)),
        pl.BlockSpec((16, D), lambda i: (0, 0)),
        pl.BlockSpec((1, D), lambda i: (0, 0)),
        pl.BlockSpec((D, R * D), lambda i: (0, 0)),
        pl.BlockSpec((R * 16, 16), lambda i: (0, 0)),
    ],
    out_specs=[
        pl.BlockSpec((NB, D), lambda i: (i, 0)),
        pl.BlockSpec((NB, R * D // 2), lambda i: (i, 0)),
        pl.BlockSpec((NB, 16), lambda i: (i, 0)),
    ],
    out_shape=[
        jax.ShapeDtypeStruct((N, D), jnp.float32),
        jax.ShapeDtypeStruct((N, R * D // 2), jnp.int32),
        jax.ShapeDtypeStruct((N, 16), jnp.float32),
    ],
)


def _dense2_body(ap_ref, h1_ref, r2_ref, b2_ref, h2_ref):
    h2_ref[...] = jnp.maximum(ap_ref[0] + ap_ref[1]
                              + h1_ref[...] @ r2_ref[...] + b2_ref[...], 0.0)


_dense2 = pl.pallas_call(
    _dense2_body,
    grid=(N // NB,),
    in_specs=[
        pl.BlockSpec((2, NB, D), lambda i: (0, i, 0)),
        pl.BlockSpec((NB, D), lambda i: (i, 0)),
        pl.BlockSpec((D, D), lambda i: (0, 0)),
        pl.BlockSpec((1, D), lambda i: (0, 0)),
    ],
    out_specs=pl.BlockSpec((NB, D), lambda i: (i, 0)),
    out_shape=jax.ShapeDtypeStruct((N, D), jnp.float32),
)


def _pool_body(h2_ref, b_ref, gw_ref, fcw_ref, fcb_ref, out_ref):
    h2 = h2_ref[...]
    gate = h2 @ gw_ref[...]
    gate = gate[:, 0:1]
    onehot = (b_ref[...] == lax.broadcasted_iota(jnp.int32, (1, G), 1))
    oh = onehot.astype(jnp.float32)
    gmax = jnp.max(jnp.where(onehot, gate, -jnp.inf), axis=0, keepdims=True)
    gmaxn = jnp.sum(oh * gmax, axis=1, keepdims=True)
    ge = jnp.exp(gate - gmaxn)
    denom = jnp.sum(oh * ge, axis=0, keepdims=True)
    denomn = jnp.sum(oh * denom, axis=1, keepdims=True)
    w = ge / denomn
    pooled = lax.dot_general(oh, h2 * w, (((0,), (0,)), ((), ())))
    out_ref[...] = pooled @ fcw_ref[...] + fcb_ref[...]


_pool = pl.pallas_call(
    _pool_body,
    out_shape=jax.ShapeDtypeStruct((G, D), jnp.float32),
)


def kernel(x, edge_index, edge_type, batch, W1, root1, b1, W2, root2, b2,
           gate_W, gate_b, fc_W, fc_b):
    src = edge_index[0]
    dst = edge_index[1]
    seg = dst * R + edge_type
    npad = EPAD - E

    xpad = jnp.zeros((N + 1, 16), jnp.float32)
    xpad = xpad.at[:N, :DIN].set(x).at[:N, DIN].set(1.0)

    src_p = jnp.concatenate([src, jnp.full((npad,), N, jnp.int32)])
    seg_p = jnp.concatenate([seg, jnp.zeros((npad,), jnp.int32)])
    zeros_a = jnp.zeros((NRP, 16), jnp.float32)

    s2 = _edge_sums1(xpad, src_p, seg_p.reshape(NW * KCH, CH), zeros_a)
    sparts = s2[:, :NR].reshape(2, N, R * 16)

    i144 = jnp.arange(R * 16)
    P = (i144[:, None] == ((i144 // 16) * 16 + DIN)[None, :]).astype(jnp.float32)
    Qp = (i144[:, None] == (jnp.arange(16) * 16 + DIN)[None, :]).astype(jnp.float32)
    W1p = jnp.zeros((R, 16, D), jnp.float32).at[:, :DIN, :].set(W1)
    root1p = jnp.zeros((16, D), jnp.float32).at[:DIN, :].set(root1)
    W2f = jnp.transpose(W2, (1, 0, 2)).reshape(D, R * D)

    h1, t2n, ict = _dense1(sparts, xpad[:N], P, W1p.reshape(R * 16, D),
                           root1p, b1[None], W2f, Qp)
    t2 = t2n.reshape(NR, D // 2)

    ic_pad = jnp.concatenate([ict[:, :R].reshape(NR),
                              jnp.zeros((16,), jnp.float32)])
    aseg_p = jnp.concatenate([seg, jnp.full((npad,), NR, jnp.int32)])
    alpha = _edge_alpha(ic_pad, aseg_p)

    tix_p = jnp.concatenate([src * R + edge_type, jnp.zeros((npad,), jnp.int32)])
    dst_p = jnp.concatenate([dst, jnp.zeros((npad,), jnp.int32)])
    zeros_c = jnp.zeros((NP, D), jnp.float32)

    a2 = _edge_sums2(t2, tix_p.reshape(NW * KC2, CH2),
                     dst_p.reshape(NW, KC2, CH2), alpha, zeros_c)

    kd = jnp.arange(D)
    kk = kd // 32
    ii = kd % 32
    perm = jnp.where(ii < 16, 16 * kk + ii, 16 * kk + ii - 16 + D // 2)
    h2 = _dense2(a2[:, :N], h1, root2[:, perm], b2[perm][None])

    gWp = jnp.zeros((D, 8), jnp.float32).at[:, 0].set(gate_W[perm, 0])
    return _pool(h2, batch.reshape(N, 1), gWp, fc_W[perm, :], fc_b[None])
